# baseline (speedup 1.0000x reference)
import jax
import jax.numpy as jnp
from jax.experimental import pallas as pl
from jax.experimental.pallas import tpu as pltpu


def _pick_block(n, candidates):
    for c in candidates:
        if n % c == 0:
            return c
    return n


def _body(adj_ab, adj_ba, xa, xb, wp_a, wself_a, wsh_a, wc_a, b_a,
          wp_b, wself_b, wsh_b, wc_b, b_b, out_a, out_b, ya_s, yb_s):
    m = pl.program_id(0)
    half = pl.num_programs(0) // 2
    bm = out_a.shape[0]
    d_out = wsh_a.shape[0]

    @pl.when(m == 0)
    def _compute_y():
        f_a = jnp.dot(jnp.dot(wp_a[...], wsh_a[...]), wc_a[:d_out, :],
                      preferred_element_type=jnp.float32)
        f_b = jnp.dot(jnp.dot(wp_b[...], wsh_b[...]), wc_b[:d_out, :],
                      preferred_element_type=jnp.float32)
        ya_s[...] = jnp.dot(xb[...], f_a, preferred_element_type=jnp.float32)
        yb_s[...] = jnp.dot(xa[...], f_b, preferred_element_type=jnp.float32)

    @pl.when(m < half)
    def _dir_a():
        g_a = jnp.dot(jnp.dot(wself_a[...], wsh_a[...]), wc_a[d_out:, :],
                      preferred_element_type=jnp.float32)
        xa_blk = xa[pl.ds(m * bm, bm), :]
        out_a[...] = (jnp.dot(adj_ab[...], ya_s[...],
                              preferred_element_type=jnp.float32)
                      + jnp.dot(xa_blk, g_a,
                                preferred_element_type=jnp.float32)
                      + b_a[...])

    @pl.when(m >= half)
    def _dir_b():
        g_b = jnp.dot(jnp.dot(wself_b[...], wsh_b[...]), wc_b[d_out:, :],
                      preferred_element_type=jnp.float32)
        xb_blk = xb[pl.ds((m - half) * bm, bm), :]
        out_b[...] = (jnp.dot(adj_ba[...], yb_s[...],
                              preferred_element_type=jnp.float32)
                      + jnp.dot(xb_blk, g_b,
                                preferred_element_type=jnp.float32)
                      + b_b[...])


def kernel(x_a, x_b, adj_a_b, adj_b_a, W_proj_a_b, w_self_a, w_share_a,
           w_att_a, w_cat_a, bias_a, W_proj_b_a, w_self_b, w_share_b,
           w_att_b, w_cat_b, bias_b):
    n, d_in = x_a.shape
    d_out = w_share_a.shape[0]

    bm = _pick_block(n, (200, 40, 8))
    nb = n // bm
    grid = (2 * nb,)

    ab_spec = pl.BlockSpec((bm, n), lambda m: (jnp.minimum(m, nb - 1), 0))
    ba_spec = pl.BlockSpec((bm, n), lambda m: (jnp.maximum(m - nb, 0), 0))
    full = lambda shape: pl.BlockSpec(shape, lambda m: (0,) * len(shape))
    oa_spec = pl.BlockSpec((bm, d_out), lambda m: (jnp.minimum(m, nb - 1), 0))
    ob_spec = pl.BlockSpec((bm, d_out), lambda m: (jnp.maximum(m - nb, 0), 0))

    out_a, out_b = pl.pallas_call(
        _body,
        grid=grid,
        in_specs=[
            ab_spec, ba_spec,
            full((n, d_in)), full((n, d_in)),
            full((d_in, d_out)), full((d_in, d_out)),
            full((d_out, d_out)), full((2 * d_out, d_out)),
            full((1, d_out)),
            full((d_in, d_out)), full((d_in, d_out)),
            full((d_out, d_out)), full((2 * d_out, d_out)),
            full((1, d_out)),
        ],
        out_specs=[oa_spec, ob_spec],
        out_shape=[jax.ShapeDtypeStruct((n, d_out), jnp.float32)] * 2,
        scratch_shapes=[pltpu.VMEM((n, d_out), jnp.float32)] * 2,
        compiler_params=pltpu.CompilerParams(
            dimension_semantics=("arbitrary",)),
    )(adj_a_b, adj_b_a, x_a, x_b,
      W_proj_a_b, w_self_a, w_share_a, w_cat_a, bias_a,
      W_proj_b_a, w_self_b, w_share_b, w_cat_b, bias_b)

    return (out_a, out_b)


# speedup vs baseline: 1.0657x; 1.0657x over previous
import jax
import jax.numpy as jnp
from jax.experimental import pallas as pl
from jax.experimental.pallas import tpu as pltpu


def _pick_block(n, candidates):
    for c in candidates:
        if n % c == 0:
            return c
    return n


def _body(adj_ab, adj_ba, xa, xb, wp_a, wself_a, wsh_a, wc_a, b_a,
          wp_b, wself_b, wsh_b, wc_b, b_b, out_a, out_b, ya_s, yb_s):
    m = pl.program_id(0)
    bm = out_a.shape[0]
    d_out = wsh_a.shape[0]

    @pl.when(m == 0)
    def _compute_y():
        f_a = jnp.dot(jnp.dot(wp_a[...], wsh_a[...]), wc_a[:d_out, :],
                      preferred_element_type=jnp.float32)
        f_b = jnp.dot(jnp.dot(wp_b[...], wsh_b[...]), wc_b[:d_out, :],
                      preferred_element_type=jnp.float32)
        ya_s[...] = jnp.dot(xb[...], f_a, preferred_element_type=jnp.float32)
        yb_s[...] = jnp.dot(xa[...], f_b, preferred_element_type=jnp.float32)

    g_a = jnp.dot(jnp.dot(wself_a[...], wsh_a[...]), wc_a[d_out:, :],
                  preferred_element_type=jnp.float32)
    g_b = jnp.dot(jnp.dot(wself_b[...], wsh_b[...]), wc_b[d_out:, :],
                  preferred_element_type=jnp.float32)
    xa_blk = xa[pl.ds(m * bm, bm), :]
    xb_blk = xb[pl.ds(m * bm, bm), :]
    out_a[...] = (jnp.dot(adj_ab[...], ya_s[...],
                          preferred_element_type=jnp.float32)
                  + jnp.dot(xa_blk, g_a, preferred_element_type=jnp.float32)
                  + b_a[...])
    out_b[...] = (jnp.dot(adj_ba[...], yb_s[...],
                          preferred_element_type=jnp.float32)
                  + jnp.dot(xb_blk, g_b, preferred_element_type=jnp.float32)
                  + b_b[...])


def kernel(x_a, x_b, adj_a_b, adj_b_a, W_proj_a_b, w_self_a, w_share_a,
           w_att_a, w_cat_a, bias_a, W_proj_b_a, w_self_b, w_share_b,
           w_att_b, w_cat_b, bias_b):
    n, d_in = x_a.shape
    d_out = w_share_a.shape[0]

    bm = _pick_block(n, (200, 40, 8))
    grid = (n // bm,)

    adj_spec = pl.BlockSpec((bm, n), lambda m: (m, 0))
    full = lambda shape: pl.BlockSpec(shape, lambda m: (0,) * len(shape))
    o_spec = pl.BlockSpec((bm, d_out), lambda m: (m, 0))

    out_a, out_b = pl.pallas_call(
        _body,
        grid=grid,
        in_specs=[
            adj_spec, adj_spec,
            full((n, d_in)), full((n, d_in)),
            full((d_in, d_out)), full((d_in, d_out)),
            full((d_out, d_out)), full((2 * d_out, d_out)),
            full((1, d_out)),
            full((d_in, d_out)), full((d_in, d_out)),
            full((d_out, d_out)), full((2 * d_out, d_out)),
            full((1, d_out)),
        ],
        out_specs=[o_spec, o_spec],
        out_shape=[jax.ShapeDtypeStruct((n, d_out), jnp.float32)] * 2,
        scratch_shapes=[pltpu.VMEM((n, d_out), jnp.float32)] * 2,
        compiler_params=pltpu.CompilerParams(
            dimension_semantics=("arbitrary",)),
    )(adj_a_b, adj_b_a, x_a, x_b,
      W_proj_a_b, w_self_a, w_share_a, w_cat_a, bias_a,
      W_proj_b_a, w_self_b, w_share_b, w_cat_b, bias_b)

    return (out_a, out_b)
